# baseline (speedup 1.0000x reference)
import functools

import jax
import jax.numpy as jnp
from jax import lax
from jax.experimental import pallas as pl
from jax.experimental.pallas import tpu as pltpu
from jax.experimental.pallas import tpu_sc as plsc

_FEATS = (160, 160, 128, 128)


def _ll_kernel(sp_ref, y_ref, wct_ref, out_ref, *, n_elem, inv_nets):
    g = pl.program_id(0)

    @pl.when(g == 0)
    def _init():
        out_ref[...] = jnp.zeros_like(out_ref)

    acc = jnp.zeros_like(out_ref)
    for j in range(y_ref.shape[0]):
        ya = y_ref[j]
        d = jnp.dot(wct_ref[...], ya,
                    preferred_element_type=jnp.float32)
        spa = sp_ref[j]
        eidx = lax.broadcasted_iota(jnp.int32, d.shape, 0)
        onehot = (spa == eidx).astype(jnp.float32)
        acc = acc + jnp.sum(onehot * (d * inv_nets), axis=0, keepdims=True)
    out_ref[...] += acc


def _sae_body(spt_hbm, cb_hbm, out_hbm, sp_v, cb_v, st_v, *, n_atoms, n_elem,
              mols_per_worker, num_cores):
    wid = lax.axis_index("s") * num_cores + lax.axis_index("c")
    base = wid * mols_per_worker
    pltpu.sync_copy(cb_hbm, cb_v)
    pltpu.sync_copy(spt_hbm, sp_v)
    for g in range(mols_per_worker // 16):
        acc = jnp.zeros((16,), jnp.float32)
        for a in range(n_atoms):
            sp = sp_v[a, pl.ds(base + g * 16, 16)]
            val = cb_v[n_elem - 1]
            for e in range(n_elem - 2, -1, -1):
                val = jnp.where(sp == e, cb_v[e], val)
            acc = acc + val
        st_v[pl.ds(g * 16, 16)] = acc
    pltpu.sync_copy(st_v, out_hbm.at[pl.ds(base, mols_per_worker)])


@jax.jit
def kernel(species, y, W, b, self_energies):
    B, A, NETS, F = y.shape
    N_ELEM = W.shape[1]
    KF = NETS * F

    feats = jnp.asarray(_FEATS[:N_ELEM], dtype=jnp.int32)
    fmask = (jnp.arange(F, dtype=jnp.int32)[None, :] < feats[:, None])
    Wm = W * fmask[None, :, :].astype(W.dtype)
    wct = Wm.transpose(1, 0, 2).reshape(N_ELEM, KF)
    csc_b = jnp.broadcast_to(
        (b.mean(axis=0) + self_energies)[:, None], (N_ELEM, 16))

    yt = jnp.transpose(y, (1, 2, 3, 0)).reshape(A, KF, B)
    spt3 = jnp.transpose(species, (1, 0)).reshape(A, 1, B)
    spt2 = jnp.transpose(species, (1, 0))

    AB = 2
    dot_part = pl.pallas_call(
        functools.partial(_ll_kernel, n_elem=N_ELEM, inv_nets=1.0 / NETS),
        grid=(A // AB,),
        in_specs=[
            pl.BlockSpec((AB, 1, B), lambda g: (g, 0, 0)),
            pl.BlockSpec((AB, KF, B), lambda g: (g, 0, 0)),
            pl.BlockSpec((N_ELEM, KF), lambda g: (0, 0)),
        ],
        out_specs=pl.BlockSpec((1, B), lambda g: (0, 0)),
        out_shape=jax.ShapeDtypeStruct((1, B), jnp.float32),
        compiler_params=pltpu.CompilerParams(
            dimension_semantics=("arbitrary",)),
    )(spt3, yt, wct)

    MW = 32
    sae = pl.kernel(
        functools.partial(_sae_body, n_atoms=A, n_elem=N_ELEM,
                          mols_per_worker=MW, num_cores=2),
        mesh=plsc.VectorSubcoreMesh(core_axis_name="c", subcore_axis_name="s"),
        out_type=jax.ShapeDtypeStruct((B,), jnp.float32),
        scratch_types=[
            pltpu.VMEM((A, B), jnp.int32),
            pltpu.VMEM((N_ELEM, 16), jnp.float32),
            pltpu.VMEM((MW,), jnp.float32),
        ],
    )(spt2, csc_b)

    return (species, dot_part.reshape(B) + sae)


# speedup vs baseline: 1.3514x; 1.3514x over previous
import functools

import jax
import jax.numpy as jnp
from jax.experimental import pallas as pl
from jax.experimental.pallas import tpu as pltpu

_FEATS = (160, 160, 128, 128)


def _ll_kernel(sp_ref, y_ref, wct_ref, c_ref, out_ref, *, n_elem, inv_nets):
    g = pl.program_id(0)

    @pl.when(g == 0)
    def _init():
        out_ref[...] = jnp.zeros_like(out_ref)

    acc = jnp.zeros_like(out_ref)
    for j in range(y_ref.shape[0]):
        ya = y_ref[j]
        d = jnp.dot(wct_ref[...], ya,
                    preferred_element_type=jnp.float32)
        spa = sp_ref[:, y_ref.shape[0] * g + j, :]
        eidx = jax.lax.broadcasted_iota(jnp.int32, d.shape, 0)
        onehot = (spa == eidx).astype(jnp.float32)
        acc = acc + jnp.sum(onehot * ((d + c_ref[...]) * inv_nets), axis=0,
                            keepdims=True)
    out_ref[...] += acc


@jax.jit
def kernel(species, y, W, b, self_energies):
    B, A, NETS, F = y.shape
    N_ELEM = W.shape[1]
    KF = NETS * F

    feats = jnp.asarray(_FEATS[:N_ELEM], dtype=jnp.int32)
    fmask = (jnp.arange(F, dtype=jnp.int32)[None, :] < feats[:, None])
    Wm = W * fmask[None, :, :].astype(W.dtype)
    wct = Wm.transpose(1, 0, 2).reshape(N_ELEM, KF)
    c = (b.sum(axis=0) + self_energies * NETS)[:, None]

    yt = jnp.transpose(y, (1, 2, 3, 0)).reshape(A, KF, B)
    spt = jnp.transpose(species, (1, 0)).reshape(1, A, B)

    AB = 2
    out = pl.pallas_call(
        functools.partial(_ll_kernel, n_elem=N_ELEM, inv_nets=1.0 / NETS),
        grid=(A // AB,),
        in_specs=[
            pl.BlockSpec((1, A, B), lambda g: (0, 0, 0)),
            pl.BlockSpec((AB, KF, B), lambda g: (g, 0, 0)),
            pl.BlockSpec((N_ELEM, KF), lambda g: (0, 0)),
            pl.BlockSpec((N_ELEM, 1), lambda g: (0, 0)),
        ],
        out_specs=pl.BlockSpec((1, B), lambda g: (0, 0)),
        out_shape=jax.ShapeDtypeStruct((1, B), jnp.float32),
        compiler_params=pltpu.CompilerParams(
            dimension_semantics=("arbitrary",)),
    )(spt, yt, wct, c)

    return (species, out.reshape(B))
